# baseline (speedup 1.0000x reference)
import functools

import jax
import jax.numpy as jnp
from jax import lax
from jax.experimental import pallas as pl
from jax.experimental.pallas import tpu as pltpu
from jax.experimental.pallas import tpu_sc as plsc

VOCAB = 100000
D = 128
HID = 512
ODIM = 128
B = 4096
T = 50

NC = 2
NS = 16
NW = NC * NS
BPW = B // NW
L = 16
DV = D // L
K = 2
NSTEP = BPW // K
NBUF = 4

_mesh = plsc.VectorSubcoreMesh(core_axis_name="c", subcore_axis_name="s")


@functools.partial(
    pl.kernel,
    mesh=_mesh,
    out_type=jax.ShapeDtypeStruct((NW, T, D), jnp.float32),
    scratch_types=[
        pltpu.VMEM((NSTEP, K * T), jnp.int32),
        pltpu.VMEM((NBUF, K * T, D), jnp.float32),
        pltpu.VMEM((T, D), jnp.float32),
        pltpu.SemaphoreType.DMA,
        pltpu.SemaphoreType.DMA,
        pltpu.SemaphoreType.DMA,
        pltpu.SemaphoreType.DMA,
    ],
)
def _sc_gather_sum(x_hbm, table_hbm, out_hbm, slab_v, ring, acc_v, *sems):
    wid = lax.axis_index("s") * NC + lax.axis_index("c")

    pltpu.sync_copy(x_hbm.at[wid], slab_v)

    def fire(s, b):
        pltpu.async_copy(table_hbm.at[slab_v.at[s]], ring.at[b], sems[b])

    def wait(s, b):
        pltpu.make_async_copy(table_hbm.at[slab_v.at[s]], ring.at[b], sems[b]).wait()

    def accumulate(s, b):
        buf = ring.at[b]

        def body(t, _):
            for k in range(DV):
                sl = pl.ds(k * L, L)
                v = buf[t, sl] + buf[T + t, sl]
                acc_v[t, sl] += v
            return 0

        lax.fori_loop(0, T, body, 0, unroll=2)

    for b in range(NBUF - 1):
        fire(b, b)

    zeros = jnp.zeros((L,), jnp.float32)

    def zero_body(t, _):
        for k in range(DV):
            acc_v[t, pl.ds(k * L, L)] = zeros
        return 0

    lax.fori_loop(0, T, zero_body, 0, unroll=2)

    def chunk(i, _):
        g = NBUF * i
        for b in range(NBUF):
            s = g + b

            @pl.when(s + NBUF - 1 < NSTEP)
            def _():
                fire(s + NBUF - 1, (b + NBUF - 1) % NBUF)

            wait(s, b)
            accumulate(s, b)
        return 0

    lax.fori_loop(0, NSTEP // NBUF, chunk, 0)

    pltpu.sync_copy(acc_v, out_hbm.at[wid])


def _mlp_body(p_ref, w1_ref, b1_ref, w2_ref, b2_ref, o_ref):
    xhat = jnp.sum(p_ref[...], axis=0) * (1.0 / B)
    h = jnp.dot(xhat, w1_ref[...], preferred_element_type=jnp.float32)
    h = jnp.maximum(h + b1_ref[...], 0.0)
    o_ref[...] = jnp.dot(h, w2_ref[...], preferred_element_type=jnp.float32) + b2_ref[...]


def kernel(x, embeddings, W1, b1, W2, b2):
    partials = _sc_gather_sum(x.astype(jnp.int32).reshape(NW, NSTEP, K * T), embeddings)
    return pl.pallas_call(
        _mlp_body,
        out_shape=jax.ShapeDtypeStruct((T, ODIM), jnp.float32),
    )(partials, W1, b1.reshape(1, HID), W2, b2.reshape(1, ODIM))


# speedup vs baseline: 1.6248x; 1.6248x over previous
import functools

import jax
import jax.numpy as jnp
from jax import lax
from jax.experimental import pallas as pl
from jax.experimental.pallas import tpu as pltpu
from jax.experimental.pallas import tpu_sc as plsc

VOCAB = 100000
D = 128
HID = 512
ODIM = 128
B = 4096
T = 50

NC = 2
NS = 16
NW = NC * NS
BPW = B // NW
L = 16
DV = D // L
NBUF = 5

_mesh = plsc.VectorSubcoreMesh(core_axis_name="c", subcore_axis_name="s")


@functools.partial(
    pl.kernel,
    mesh=_mesh,
    out_type=jax.ShapeDtypeStruct((NW, T, D), jnp.float32),
    scratch_types=[
        pltpu.VMEM((T, BPW), jnp.int32),
        pltpu.VMEM((NBUF, BPW, D), jnp.float32),
        pltpu.VMEM((T, D), jnp.float32),
        pltpu.SemaphoreType.DMA,
        pltpu.SemaphoreType.DMA,
        pltpu.SemaphoreType.DMA,
        pltpu.SemaphoreType.DMA,
        pltpu.SemaphoreType.DMA,
    ],
)
def _sc_gather_sum(xs_hbm, table_hbm, out_hbm, idx_v, ring, acc_v, *sems):
    wid = lax.axis_index("s") * NC + lax.axis_index("c")

    pltpu.sync_copy(xs_hbm.at[wid], idx_v)

    def fire(t, b):
        pltpu.async_copy(table_hbm.at[idx_v.at[t]], ring.at[b], sems[b])

    def wait(t, b):
        pltpu.make_async_copy(table_hbm.at[idx_v.at[t]], ring.at[b], sems[b]).wait()

    def accumulate(t, b):
        buf = ring.at[b]

        def body(j, carry):
            return tuple(carry[k] + buf[j, pl.ds(k * L, L)] for k in range(DV))

        init = tuple(buf[0, pl.ds(k * L, L)] for k in range(DV))
        total = lax.fori_loop(1, BPW, body, init, unroll=4)
        for k in range(DV):
            acc_v[t, pl.ds(k * L, L)] = total[k]

    for b in range(NBUF - 1):
        fire(b, b)

    def chunk(i, _):
        g = NBUF * i
        for b in range(NBUF):
            t = g + b

            @pl.when(t + NBUF - 1 < T)
            def _():
                fire(t + NBUF - 1, (b + NBUF - 1) % NBUF)

            wait(t, b)
            accumulate(t, b)
        return 0

    lax.fori_loop(0, T // NBUF, chunk, 0)

    pltpu.sync_copy(acc_v, out_hbm.at[wid])


def _mlp_body(p_ref, w1_ref, b1_ref, w2_ref, b2_ref, o_ref):
    xhat = jnp.sum(p_ref[...], axis=0) * (1.0 / B)
    h = jnp.dot(xhat, w1_ref[...], preferred_element_type=jnp.float32)
    h = jnp.maximum(h + b1_ref[...], 0.0)
    o_ref[...] = jnp.dot(h, w2_ref[...], preferred_element_type=jnp.float32) + b2_ref[...]


def kernel(x, embeddings, W1, b1, W2, b2):
    xs = x.astype(jnp.int32).reshape(NW, BPW, T).transpose(0, 2, 1)
    partials = _sc_gather_sum(xs, embeddings)
    return pl.pallas_call(
        _mlp_body,
        out_shape=jax.ShapeDtypeStruct((T, ODIM), jnp.float32),
    )(partials, W1, b1.reshape(1, HID), W2, b2.reshape(1, ODIM))
